# baseline (speedup 1.0000x reference)
import functools

import jax
import jax.numpy as jnp
from jax import lax
from jax.experimental import pallas as pl
from jax.experimental.pallas import tpu as pltpu
from jax.experimental.pallas import tpu_sc as plsc

N_FIELDS = 26
VOCAB = 1000
EMB_DIM = 1000
BATCH = 1024
ROWS = BATCH * N_FIELDS
NC, NS, L = 2, 16, 16
NW = NC * NS
B_PER_W = BATCH // NW
XV_LEN = B_PER_W * N_FIELDS + L
IDX_PAD = 32
NSEG = 8
AUX_COL = EMB_DIM - 128
OUT_LEN = BATCH * N_FIELDS * EMB_DIM


def _make_gather():
    mesh = plsc.VectorSubcoreMesh(core_axis_name="c", subcore_axis_name="s")

    @functools.partial(
        pl.kernel,
        mesh=mesh,
        out_type=jax.ShapeDtypeStruct((OUT_LEN,), jnp.float32),
        scratch_types=[
            pltpu.VMEM((XV_LEN,), jnp.int32),
            pltpu.VMEM((B_PER_W, IDX_PAD), jnp.int32),
            pltpu.VMEM((NSEG, IDX_PAD, 128), jnp.float32),
            pltpu.VMEM((NSEG, IDX_PAD, 128), jnp.float32),
            pltpu.VMEM((N_FIELDS * EMB_DIM,), jnp.float32),
            pltpu.VMEM((N_FIELDS * EMB_DIM,), jnp.float32),
            pltpu.SemaphoreType.DMA,
            pltpu.SemaphoreType.DMA,
            pltpu.SemaphoreType.DMA,
            pltpu.SemaphoreType.DMA,
        ],
        compiler_params=pltpu.CompilerParams(use_tc_tiling_on_sc=True,
                                             needs_layout_passes=False),
    )
    def gather_kernel(table, aux, x_hbm, out, x_v, idx_v,
                      seg0, seg1, st0, st1, gsem0, gsem1, ssem0, ssem1):
        wid = lax.axis_index("s") * NC + lax.axis_index("c")
        nrows = B_PER_W * N_FIELDS
        pltpu.sync_copy(x_hbm.at[pl.ds(wid * nrows, nrows)],
                        x_v.at[pl.ds(0, nrows)])
        one16 = jnp.full((L,), 1, jnp.int32)
        x_v[pl.ds(nrows, L)] = one16

        lane = lax.broadcasted_iota(jnp.int32, (L,), 0)
        zero16i = jnp.zeros((L,), jnp.int32)
        zeros16 = jnp.zeros((L,), jnp.float32)
        cvocab = jnp.full((L,), VOCAB, jnp.int32)
        fld0 = lane * cvocab
        in1 = lane < jnp.full((L,), N_FIELDS - L, jnp.int32)
        fld1 = jnp.where(in1, (lane + jnp.full((L,), L, jnp.int32)) * cvocab,
                         zero16i)

        def idx_body(c, carry):
            xv0 = x_v[pl.ds(c * N_FIELDS, L)]
            xv1 = x_v[pl.ds(c * N_FIELDS + L, L)]
            idx_v[c, pl.ds(0, L)] = xv0 + fld0
            idx_v[c, pl.ds(L, L)] = jnp.where(in1, xv1 + fld1, zero16i)
            return carry

        lax.fori_loop(0, B_PER_W, idx_body, 0)

        segs = (seg0, seg1)
        stages = (st0, st1)
        gsems = (gsem0, gsem1)
        ssems = (ssem0, ssem1)

        def gathers(c, b, issue):
            idx_sl = idx_v.at[c]
            for s in range(NSEG - 1):
                src = table.at[idx_sl, pl.ds(s * 128, 128)]
                if issue:
                    pltpu.async_copy(src, segs[b].at[s], gsems[b])
                else:
                    pltpu.make_async_copy(src, segs[b].at[s],
                                          gsems[b]).wait()
            src = aux.at[idx_sl]
            if issue:
                pltpu.async_copy(src, segs[b].at[NSEG - 1], gsems[b])
            else:
                pltpu.make_async_copy(src, segs[b].at[NSEG - 1],
                                      gsems[b]).wait()

        def zero_pad_rows(c, b):
            for g in range(2):
                xv = x_v[pl.ds(c * N_FIELDS + g * L, L)]
                valid = lane < jnp.full((L,), N_FIELDS - g * L, jnp.int32)
                guarded = jnp.where(valid, xv, one16)
                min_x = jnp.min(guarded)

                @pl.when(min_x == 0)
                def _zero(g=g, guarded=guarded, b=b):
                    pad = guarded == zero16i
                    rows = g * L + lane

                    def body(col, carry):
                        cols = jnp.full((L,), col, jnp.int32)
                        for s in range(NSEG):
                            plsc.store_scatter(segs[b].at[s], [rows, cols],
                                               zeros16, mask=pad)
                        return carry

                    lax.fori_loop(0, 128, body, 0)

        def repack(b):
            def row_body(f, carry):
                base = f * EMB_DIM
                for s in range(NSEG):
                    off = s * 128 if s < NSEG - 1 else AUX_COL
                    for k in range(128 // L):
                        stages[b][pl.ds(base + off + k * L, L)] = (
                            segs[b][s, f, pl.ds(k * L, L)])
                return carry

            lax.fori_loop(0, N_FIELDS, row_body, 0)

        def scatter(c, b, issue):
            bg = wid * B_PER_W + c
            src = stages[b]
            dst = out.at[pl.ds(bg * (N_FIELDS * EMB_DIM),
                               N_FIELDS * EMB_DIM)]
            if issue:
                pltpu.async_copy(src, dst, ssems[b])
            else:
                pltpu.make_async_copy(src, dst, ssems[b]).wait()

        gathers(0, 0, True)
        gathers(1, 1, True)

        def chunk_body(k, carry):
            for sub in range(2):
                c = 2 * k + sub
                gathers(c, sub, False)
                zero_pad_rows(c, sub)
                repack(sub)

                @pl.when(c + 2 < B_PER_W)
                def _prefetch(c=c, sub=sub):
                    gathers(c + 2, sub, True)

                @pl.when(c >= 2)
                def _drain(c=c, sub=sub):
                    scatter(c - 2, sub, False)

                scatter(c, sub, True)
            return carry

        lax.fori_loop(0, B_PER_W // 2, chunk_body, 0)
        scatter(B_PER_W - 2, 0, False)
        scatter(B_PER_W - 1, 1, False)

    return gather_kernel


_gather = _make_gather()


def kernel(x, tables):
    table_flat = tables.reshape(N_FIELDS * VOCAB, EMB_DIM)
    aux_flat = tables[:, :, AUX_COL:].reshape(N_FIELDS * VOCAB, 128)
    x_flat = x.reshape(ROWS)
    out = _gather(table_flat, aux_flat, x_flat)
    return out.reshape(BATCH, N_FIELDS * EMB_DIM)


# speedup vs baseline: 1.9793x; 1.9793x over previous
import functools

import jax
import jax.numpy as jnp
from jax import lax
from jax.experimental import pallas as pl
from jax.experimental.pallas import tpu as pltpu
from jax.experimental.pallas import tpu_sc as plsc

N_FIELDS = 26
VOCAB = 1000
EMB_DIM = 1000
BATCH = 1024
ROWS = BATCH * N_FIELDS
NC, NS, L = 2, 16, 16
NW = NC * NS
ROWS_PER_W = ROWS // NW
CHUNK = 32
NCHUNK = ROWS_PER_W // CHUNK
NSEG = 8
TAIL_COL = (NSEG - 1) * 128
OUT_MINOR = NSEG * 128
NGROUP = ROWS_PER_W // L
HALF = BATCH // 2


def _make_gather(rows):
    mesh = plsc.VectorSubcoreMesh(core_axis_name="c", subcore_axis_name="s")

    rows_per_w = rows // NW
    nchunk = rows_per_w // CHUNK
    ngroup = rows_per_w // L

    @functools.partial(
        pl.kernel,
        mesh=mesh,
        out_type=jax.ShapeDtypeStruct((rows, OUT_MINOR), jnp.float32),
        scratch_types=[
            pltpu.VMEM((rows_per_w,), jnp.int32),
            pltpu.VMEM((rows_per_w,), jnp.int32),
            pltpu.VMEM((NSEG, CHUNK, 128), jnp.float32),
            pltpu.VMEM((NSEG, CHUNK, 128), jnp.float32),
            pltpu.SemaphoreType.DMA,
            pltpu.SemaphoreType.DMA,
            pltpu.SemaphoreType.DMA,
            pltpu.SemaphoreType.DMA,
        ],
        compiler_params=pltpu.CompilerParams(use_tc_tiling_on_sc=True,
                                             needs_layout_passes=False),
    )
    def gather_kernel(table, aux, x_hbm, out, x_v, idx_v,
                      buf0, buf1, gsem0, gsem1, ssem0, ssem1):
        wid = lax.axis_index("s") * NC + lax.axis_index("c")
        base_row = wid * rows_per_w
        pltpu.sync_copy(x_hbm.at[pl.ds(base_row, rows_per_w)], x_v)

        lane = lax.broadcasted_iota(jnp.int32, (L,), 0)
        zero16i = jnp.zeros((L,), jnp.int32)
        zeros16 = jnp.zeros((L,), jnp.float32)
        cvocab = jnp.full((L,), VOCAB, jnp.int32)
        c16 = jnp.full((L,), L, jnp.int32)
        c26 = jnp.full((L,), N_FIELDS, jnp.int32)

        def idx_body(j, fld):
            v = x_v[pl.ds(j * L, L)]
            idx_v[pl.ds(j * L, L)] = v + fld * cvocab
            t = fld + c16
            return jnp.where(t >= c26, t - c26, t)

        lax.fori_loop(0, ngroup, idx_body, lane)

        bufs = (buf0, buf1)
        gsems = (gsem0, gsem1)
        ssems = (ssem0, ssem1)

        def issue_gathers(c, b):
            off = pl.multiple_of(c * CHUNK, CHUNK)
            idx_sl = idx_v.at[pl.ds(off, CHUNK)]
            for seg in range(NSEG - 1):
                pltpu.async_copy(table.at[idx_sl, pl.ds(seg * 128, 128)],
                                 bufs[b].at[seg], gsems[b])
            pltpu.async_copy(aux.at[idx_sl], bufs[b].at[NSEG - 1], gsems[b])

        def wait_gathers(c, b):
            off = pl.multiple_of(c * CHUNK, CHUNK)
            idx_sl = idx_v.at[pl.ds(off, CHUNK)]
            for seg in range(NSEG - 1):
                pltpu.make_async_copy(
                    table.at[idx_sl, pl.ds(seg * 128, 128)],
                    bufs[b].at[seg], gsems[b]).wait()
            pltpu.make_async_copy(aux.at[idx_sl], bufs[b].at[NSEG - 1],
                                  gsems[b]).wait()

        def zero_pad_rows(c, b):
            for g in range(CHUNK // L):
                xv = x_v[pl.ds(c * CHUNK + g * L, L)]
                min_x = jnp.min(xv)

                @pl.when(min_x == 0)
                def _zero(g=g, xv=xv, b=b):
                    pad = xv == zero16i
                    rows = g * L + lane

                    def body(col, carry):
                        cols = jnp.full((L,), col, jnp.int32)
                        for seg in range(NSEG):
                            plsc.store_scatter(bufs[b].at[seg], [rows, cols],
                                               zeros16, mask=pad)
                        return carry

                    lax.fori_loop(0, 128, body, 0)

        def scatters(c, b, issue):
            r0 = pl.multiple_of(base_row + c * CHUNK, CHUNK)
            for seg in range(NSEG):
                src = bufs[b].at[seg]
                dst = out.at[pl.ds(r0, CHUNK), pl.ds(seg * 128, 128)]
                if issue:
                    pltpu.async_copy(src, dst, ssems[b])
                else:
                    pltpu.make_async_copy(src, dst, ssems[b]).wait()

        issue_gathers(0, 0)
        issue_gathers(1, 1)

        def chunk_body(k, carry):
            for sub in range(2):
                c = 2 * k + sub
                wait_gathers(c, sub)
                zero_pad_rows(c, sub)
                scatters(c, sub, True)
                scatters(c, sub, False)

                @pl.when(c + 2 < nchunk)
                def _prefetch(c=c, sub=sub):
                    issue_gathers(c + 2, sub)

            return carry

        lax.fori_loop(0, nchunk // 2, chunk_body, 0)
        if nchunk % 2:
            c = nchunk - 1
            sub = c % 2
            wait_gathers(c, sub)
            zero_pad_rows(c, sub)
            scatters(c, sub, True)
            scatters(c, sub, False)

    return gather_kernel


_gather_half = _make_gather(HALF * N_FIELDS)


def kernel(x, tables):
    table_flat = tables.reshape(N_FIELDS * VOCAB, EMB_DIM)
    aux = jnp.pad(tables[:, :, TAIL_COL:],
                  ((0, 0), (0, 0), (0, OUT_MINOR - EMB_DIM)))
    aux_flat = aux.reshape(N_FIELDS * VOCAB, 128)
    x_flat = x.reshape(ROWS)
    h = HALF * N_FIELDS
    out0 = _gather_half(table_flat, aux_flat, x_flat[:h])
    out1 = _gather_half(table_flat, aux_flat, x_flat[h:])
    out = jnp.concatenate([out0, out1], axis=0)
    return out[:, :EMB_DIM].reshape(BATCH, N_FIELDS * EMB_DIM)


# speedup vs baseline: 2.5361x; 1.2813x over previous
import functools

import jax
import jax.numpy as jnp
from jax import lax
from jax.experimental import pallas as pl
from jax.experimental.pallas import tpu as pltpu
from jax.experimental.pallas import tpu_sc as plsc

N_FIELDS = 26
VOCAB = 1000
EMB_DIM = 1000
BATCH = 1024
ROWS = BATCH * N_FIELDS
NC, NS, L = 2, 16, 16
NW = NC * NS
ROWS_PER_W = ROWS // NW
CHUNK = 32
NCHUNK = ROWS_PER_W // CHUNK
NSEG = 8
TAIL_COL = (NSEG - 1) * 128
OUT_MINOR = NSEG * 128
NGROUP = ROWS_PER_W // L


def _make_gather():
    mesh = plsc.VectorSubcoreMesh(core_axis_name="c", subcore_axis_name="s")

    @functools.partial(
        pl.kernel,
        mesh=mesh,
        out_type=jax.ShapeDtypeStruct((ROWS, OUT_MINOR), jnp.float32),
        scratch_types=[
            pltpu.VMEM((ROWS_PER_W,), jnp.int32),
            pltpu.VMEM((ROWS_PER_W,), jnp.int32),
            pltpu.VMEM((NSEG, CHUNK, 128), jnp.float32),
            pltpu.VMEM((NSEG, CHUNK, 128), jnp.float32),
            pltpu.SemaphoreType.DMA,
            pltpu.SemaphoreType.DMA,
            pltpu.SemaphoreType.DMA,
            pltpu.SemaphoreType.DMA,
        ],
        compiler_params=pltpu.CompilerParams(use_tc_tiling_on_sc=True,
                                             needs_layout_passes=False),
    )
    def gather_kernel(table, aux, x_hbm, out, x_v, idx_v,
                      buf0, buf1, gsem0, gsem1, ssem0, ssem1):
        wid = lax.axis_index("s") * NC + lax.axis_index("c")
        base_row = wid * ROWS_PER_W
        pltpu.sync_copy(x_hbm.at[pl.ds(base_row, ROWS_PER_W)], x_v)

        lane = lax.broadcasted_iota(jnp.int32, (L,), 0)
        zero16i = jnp.zeros((L,), jnp.int32)
        zeros16 = jnp.zeros((L,), jnp.float32)
        cvocab = jnp.full((L,), VOCAB, jnp.int32)
        c16 = jnp.full((L,), L, jnp.int32)
        c26 = jnp.full((L,), N_FIELDS, jnp.int32)

        def idx_body(j, fld):
            v = x_v[pl.ds(j * L, L)]
            idx_v[pl.ds(j * L, L)] = v + fld * cvocab
            t = fld + c16
            return jnp.where(t >= c26, t - c26, t)

        lax.fori_loop(0, NGROUP, idx_body, lane)

        bufs = (buf0, buf1)
        gsems = (gsem0, gsem1)
        ssems = (ssem0, ssem1)

        def issue_gathers(c, b):
            off = pl.multiple_of(c * CHUNK, CHUNK)
            idx_sl = idx_v.at[pl.ds(off, CHUNK)]
            for seg in range(NSEG - 1):
                pltpu.async_copy(table.at[idx_sl, pl.ds(seg * 128, 128)],
                                 bufs[b].at[seg], gsems[b])
            pltpu.async_copy(aux.at[idx_sl], bufs[b].at[NSEG - 1], gsems[b])

        def wait_gathers(c, b):
            off = pl.multiple_of(c * CHUNK, CHUNK)
            idx_sl = idx_v.at[pl.ds(off, CHUNK)]
            for seg in range(NSEG - 1):
                pltpu.make_async_copy(
                    table.at[idx_sl, pl.ds(seg * 128, 128)],
                    bufs[b].at[seg], gsems[b]).wait()
            pltpu.make_async_copy(aux.at[idx_sl], bufs[b].at[NSEG - 1],
                                  gsems[b]).wait()

        def zero_pad_rows(c, b):
            for g in range(CHUNK // L):
                xv = x_v[pl.ds(c * CHUNK + g * L, L)]
                min_x = jnp.min(xv)

                @pl.when(min_x == 0)
                def _zero(g=g, xv=xv, b=b):
                    pad = xv == zero16i
                    rows = g * L + lane

                    def body(col, carry):
                        cols = jnp.full((L,), col, jnp.int32)
                        for seg in range(NSEG):
                            plsc.store_scatter(bufs[b].at[seg], [rows, cols],
                                               zeros16, mask=pad)
                        return carry

                    lax.fori_loop(0, 128, body, 0)

        def scatters(c, b, issue):
            r0 = pl.multiple_of(base_row + c * CHUNK, CHUNK)
            for seg in range(NSEG):
                src = bufs[b].at[seg]
                dst = out.at[pl.ds(r0, CHUNK), pl.ds(seg * 128, 128)]
                if issue:
                    pltpu.async_copy(src, dst, ssems[b])
                else:
                    pltpu.make_async_copy(src, dst, ssems[b]).wait()

        issue_gathers(0, 0)
        issue_gathers(1, 1)

        def chunk_body(k, carry):
            for sub in range(2):
                c = 2 * k + sub
                wait_gathers(c, sub)
                zero_pad_rows(c, sub)
                scatters(c, sub, True)
                scatters(c, sub, False)

                @pl.when(c + 2 < NCHUNK)
                def _prefetch(c=c, sub=sub):
                    issue_gathers(c + 2, sub)

            return carry

        lax.fori_loop(0, NCHUNK // 2, chunk_body, 0)

    return gather_kernel


_gather = _make_gather()


def kernel(x, tables):
    table_flat = tables.reshape(N_FIELDS * VOCAB, EMB_DIM)
    aux = jnp.pad(tables[:, :, TAIL_COL:],
                  ((0, 0), (0, 0), (0, OUT_MINOR - EMB_DIM)))
    aux_flat = aux.reshape(N_FIELDS * VOCAB, 128)
    x_flat = x.reshape(ROWS)
    out = _gather(table_flat, aux_flat, x_flat)
    return out[:, :EMB_DIM].reshape(BATCH, N_FIELDS * EMB_DIM)
